# baseline (speedup 1.0000x reference)
import functools

import jax
import jax.numpy as jnp
from jax import lax
from jax.experimental import pallas as pl
from jax.experimental.pallas import tpu as pltpu
from jax.experimental.pallas import tpu_sc as plsc

DIM = 64

_info = plsc.get_sparse_core_info()
_NC, _NS = _info.num_cores, _info.num_subcores
_NW = _NC * _NS


@functools.cache
def _make(R: int, S: int, CHR: int, D: int):
    assert R % (_NW * CHR) == 0
    r_per_w = R // _NW
    n_ch = r_per_w // CHR
    assert n_ch % D == 0 and n_ch >= 2 * D
    mesh = plsc.VectorSubcoreMesh(core_axis_name="c", subcore_axis_name="s")

    @functools.partial(
        pl.kernel,
        out_type=jax.ShapeDtypeStruct((R, S, DIM), jnp.float32),
        mesh=mesh,
        scratch_types=[
            pltpu.VMEM((r_per_w, S), jnp.int32),
            [pltpu.VMEM((S, DIM), jnp.float32) for _ in range(D)],
            [pltpu.SemaphoreType.DMA for _ in range(D)],
            [pltpu.SemaphoreType.DMA for _ in range(D)],
        ],
        compiler_params=pltpu.CompilerParams(use_tc_tiling_on_sc=False),
    )
    def gather_kernel(idx_hbm, table_hbm, out_hbm, idx_v, rows, sg, sw):
        wid = lax.axis_index("s") * _NC + lax.axis_index("c")
        base = wid * r_per_w
        pltpu.sync_copy(idx_hbm.at[pl.ds(base, r_per_w), :], idx_v)

        def gather(tc, b):
            return pltpu.async_copy(
                table_hbm.at[idx_v.at[tc]],
                rows[b], sg[b])

        def gather_wait(tc, b):
            pltpu.make_async_copy(
                table_hbm.at[idx_v.at[tc]],
                rows[b], sg[b]).wait()

        def write(tc, b):
            return pltpu.async_copy(
                rows[b], out_hbm.at[base + tc], sw[b])

        def write_wait(tc, b):
            pltpu.make_async_copy(
                rows[b], out_hbm.at[base + tc], sw[b]).wait()

        for b in range(D):
            gather(b, b)

        @pl.loop(0, n_ch - D, step=D)
        def _(t):
            for b in range(D):
                tc = t + b
                gather_wait(tc, b)
                write(tc, b)
                write_wait(tc, b)
                gather(tc + D, b)

        for b in range(D):
            tc = n_ch - D + b
            gather_wait(tc, b)
            write(tc, b)
        for b in range(D):
            write_wait(n_ch - D + b, b)

    return gather_kernel


def kernel(inputs, embedding_table):
    R, S = inputs.shape
    idx = inputs.astype(jnp.int32)
    return _make(R, S, 1, 8)(idx, embedding_table)
